# baseline (speedup 1.0000x reference)
import dataclasses
import functools

import jax
import jax.numpy as jnp
from jax import lax
from jax.experimental import pallas as pl
from jax.experimental.pallas import tpu as pltpu
from jax.experimental.pallas import tpu_sc as plsc

N = 10000
D = 128
E = 320000
NC = 2
NS = 16
NW = NC * NS
K = 128
EPW = 10240
NB = EPW // K
NB_G = NB + 2
E_PAD = NW * EPW
ACC_N = 10240
ZROWS = ACC_N // NS
SHIFT = 14
MASK = (1 << SHIFT) - 1
L = 16

_mesh = plsc.VectorSubcoreMesh(core_axis_name="c", subcore_axis_name="s")

_cp = pltpu.CompilerParams()
if "needs_layout_passes" in pltpu.CompilerParams.__dataclass_fields__:
    _cp = dataclasses.replace(_cp, needs_layout_passes=False)

@functools.partial(
    pl.kernel,
    out_type=jax.ShapeDtypeStruct((NC, NS, ACC_N), jnp.float32),
    mesh=_mesh,
    compiler_params=_cp,
    scratch_types=[
        pltpu.VMEM((EPW,), jnp.int32),
        pltpu.VMEM((ACC_N,), jnp.float32),
    ],
)
def _sc_degree(pkf_hbm, zeros_hbm, out_hbm, pk_v, hist_v):
    c = lax.axis_index("c")
    s = lax.axis_index("s")
    wid = c * NS + s
    pltpu.sync_copy(pkf_hbm.at[wid], pk_v)
    pltpu.sync_copy(zeros_hbm, hist_v)
    ones_vec = jnp.full((L,), 1.0, jnp.float32)

    @pl.loop(0, EPW, step=L)
    def _(i):
        d = lax.bitwise_and(pk_v[pl.ds(i, L)], MASK)
        plsc.addupdate_scatter(hist_v, [d], ones_vec)

    pltpu.sync_copy(hist_v, out_hbm.at[c, s])


@functools.partial(
    pl.kernel,
    out_type=jax.ShapeDtypeStruct((NC, ACC_N, D), jnp.float32),
    mesh=_mesh,
    scratch_types=[
        pltpu.VMEM((NB_G, K), jnp.int32),
        pltpu.VMEM((4, K), jnp.int32),
        pltpu.VMEM((K, D), jnp.float32),
        pltpu.VMEM((K, D), jnp.float32),
        pltpu.VMEM_SHARED((ACC_N, D), jnp.float32),
        pltpu.SemaphoreType.DMA,
        pltpu.SemaphoreType.DMA,
    ],
)
def _sc_edge(h_hbm, pk_hbm, zeros_hbm, out_hbm,
             pk_v, ring, rows0, rows1, acc_sh, sem0, sem1):
    c = lax.axis_index("c")
    s = lax.axis_index("s")
    wid = c * NS + s

    def unpack_src(b, slot):
        @pl.loop(0, K, step=L)
        def _(j):
            ring[slot, pl.ds(j, L)] = lax.shift_right_logical(
                pk_v[b, pl.ds(j, L)], SHIFT)

    def unpack_dst(b, slot):
        @pl.loop(0, K, step=L)
        def _(j):
            ring[slot, pl.ds(j, L)] = lax.bitwise_and(pk_v[b, pl.ds(j, L)],
                                                      MASK)

    pltpu.sync_copy(pk_hbm.at[wid], pk_v)
    pltpu.sync_copy(zeros_hbm, acc_sh.at[pl.ds(s * ZROWS, ZROWS)])
    plsc.subcore_barrier()
    unpack_src(0, 0)
    pltpu.async_copy(h_hbm.at[ring.at[0]], rows0, sem0)
    unpack_src(1, 1)
    pltpu.async_copy(h_hbm.at[ring.at[1]], rows1, sem1)

    @pl.loop(0, NB, step=2)
    def _(b):
        pltpu.make_async_copy(h_hbm.at[ring.at[0]], rows0, sem0).wait()
        unpack_dst(b, 2)
        pltpu.sync_copy(rows0, acc_sh.at[ring.at[2]], add=True)
        unpack_src(b + 2, 0)
        pltpu.async_copy(h_hbm.at[ring.at[0]], rows0, sem0)
        pltpu.make_async_copy(h_hbm.at[ring.at[1]], rows1, sem1).wait()
        unpack_dst(b + 1, 3)
        pltpu.sync_copy(rows1, acc_sh.at[ring.at[3]], add=True)
        unpack_src(b + 3, 1)
        pltpu.async_copy(h_hbm.at[ring.at[1]], rows1, sem1)

    pltpu.make_async_copy(h_hbm.at[ring.at[0]], rows0, sem0).wait()
    pltpu.make_async_copy(h_hbm.at[ring.at[1]], rows1, sem1).wait()
    plsc.subcore_barrier()
    pltpu.sync_copy(
        acc_sh.at[pl.ds(s * ZROWS, ZROWS)],
        out_hbm.at[c, pl.ds(s * ZROWS, ZROWS)],
    )


def _tc_dis_body(deg_ref, dis_ref):
    dsum = jnp.sum(deg_ref[...], axis=(0, 1)) + 1.0
    dis_ref[...] = lax.rsqrt(dsum)


def _tc_prep_body(x_ref, w_ref, dis_ref, h1p_ref):
    h = lax.dot_general(x_ref[...], w_ref[...], (((1,), (0,)), ((), ())),
                        precision=lax.Precision.HIGHEST,
                        preferred_element_type=jnp.float32)
    h1p_ref[...] = h * dis_ref[...]


def _tc_mid_body(acc_ref, h1p_ref, dis_ref, b1_ref, w2_ref, h2p_ref):
    dis = dis_ref[...]
    tot = acc_ref[0, :N, :] + acc_ref[1, :N, :] + h1p_ref[...]
    z1 = jnp.maximum(tot * dis + b1_ref[...], 0.0)
    h2 = lax.dot_general(z1, w2_ref[...], (((1,), (0,)), ((), ())),
                         precision=lax.Precision.HIGHEST,
                         preferred_element_type=jnp.float32)
    h2p_ref[...] = h2 * dis


def _tc_final_body(acc_ref, h2p_ref, dis_ref, b2_ref, out_ref):
    tot = acc_ref[0, :N, :] + acc_ref[1, :N, :] + h2p_ref[...]
    out_ref[...] = tot * dis_ref[...] + b2_ref[...]


_tc_dis = pl.pallas_call(
    _tc_dis_body,
    out_shape=jax.ShapeDtypeStruct((ACC_N,), jnp.float32),
)

_tc_prep = pl.pallas_call(
    _tc_prep_body,
    out_shape=jax.ShapeDtypeStruct((N, D), jnp.float32),
)

_tc_mid = pl.pallas_call(
    _tc_mid_body,
    out_shape=jax.ShapeDtypeStruct((N, D), jnp.float32),
)

_tc_final = pl.pallas_call(
    _tc_final_body,
    out_shape=jax.ShapeDtypeStruct((N, D), jnp.float32),
)


@jax.jit
def kernel(x, edge_index, W1, b1, W2, b2):
    src = edge_index[0].astype(jnp.int32)
    dst = edge_index[1].astype(jnp.int32)
    pad = E_PAD - E
    src_p = jnp.concatenate([src, jnp.zeros((pad,), jnp.int32)])
    dst_p = jnp.concatenate(
        [dst, N + (jnp.arange(pad, dtype=jnp.int32) % (ACC_N - N))])
    packed = jnp.left_shift(src_p, SHIFT) | dst_p
    pk_g = jnp.concatenate(
        [packed.reshape(NW, NB, K), jnp.zeros((NW, 2, K), jnp.int32)], axis=1)

    zeros_d = jnp.zeros((ZROWS, D), jnp.float32)
    zeros_n = jnp.zeros((ACC_N,), jnp.float32)
    b1r = b1.reshape(1, D)
    b2r = b2.reshape(1, D)

    pk_flat = packed.reshape(NW, EPW)
    deg32 = _sc_degree(pk_flat, zeros_n)
    dis = _tc_dis(deg32)
    dis_col = dis.reshape(ACC_N, 1)[:N]
    h1p = _tc_prep(x, W1, dis_col)
    acc1 = _sc_edge(h1p, pk_g, zeros_d)
    h2p = _tc_mid(acc1, h1p, dis_col, b1r, W2)
    acc2 = _sc_edge(h2p, pk_g, zeros_d)
    return _tc_final(acc2, h2p, dis_col, b2r)
